# baseline (speedup 1.0000x reference)
import jax
import jax.numpy as jnp
from jax.experimental import pallas as pl
from jax.experimental.pallas import tpu as pltpu

_NUM_CLASSES = 80
_TEST_CONF = 0.01
_NMS_THRE = 0.65
_MAX_DET = 200
_N_PRED = 20000
_R = 160
_C = 128
_NPAD = _R * _C
_CH = 8
_NCHUNK = _R // _CH


def _comb(a, b, positional):
    gt = b[0] > a[0]
    take = gt if positional else gt | ((b[0] == a[0]) & (b[1] < a[1]))
    return tuple(jnp.where(take, fb, fa) for fa, fb in zip(a, b))


def _chunk_tournament(fields):
    pairs = [tuple(f[v * _CH:(v + 1) * _CH] for f in fields)
             for v in range(_NCHUNK)]
    while len(pairs) > 1:
        nxt = [_comb(pairs[j], pairs[j + 1], True)
               for j in range(0, len(pairs) - 1, 2)]
        if len(pairs) % 2:
            nxt.append(pairs[-1])
        pairs = nxt
    return pairs[0]


def _final_tree(t):
    h = _CH
    while h > 1:
        h //= 2
        t = _comb(tuple(f[:h] for f in t), tuple(f[h:] for f in t), True)
    wdt = _C
    while wdt > 1:
        wdt //= 2
        t = _comb(tuple(f[:, :wdt] for f in t),
                  tuple(f[:, wdt:] for f in t), False)
    return t


def _nms_body(cx_ref, cy_ref, w_ref, h_ref, obj_ref, cls_ref, out_ref,
              scores_ref, nbx1_ref, nby1_ref, nbx2_ref, nby2_ref,
              areas_ref, ox1_ref, oy1_ref, ox2_ref, oy2_ref, off_ref):
    def cls_step(k, carry):
        m, am = carry
        v = cls_ref[k]
        gt = v > m
        am = jnp.where(gt, k, am)
        m = jnp.maximum(m, v)
        return m, am

    m0 = cls_ref[0]
    am0 = jnp.zeros((_R, _C), jnp.int32)
    cls_conf, cls_pred = jax.lax.fori_loop(1, _NUM_CLASSES, cls_step,
                                           (m0, am0), unroll=8)
    conf = obj_ref[:] * cls_conf
    scores = jnp.where(conf >= _TEST_CONF, conf, 0.0)
    off = cls_pred.astype(jnp.float32) * 4096.0

    cx = cx_ref[:]
    cy = cy_ref[:]
    w = w_ref[:]
    h = h_ref[:]
    x1 = cx - w / 2.0
    y1 = cy - h / 2.0
    x2 = cx + w / 2.0
    y2 = cy + h / 2.0
    nbx1 = x1 + off
    nby1 = y1 + off
    nbx2 = x2 + off
    nby2 = y2 + off

    scores_ref[:] = scores
    nbx1_ref[:] = nbx1
    nby1_ref[:] = nby1
    nbx2_ref[:] = nbx2
    nby2_ref[:] = nby2
    areas_ref[:] = (nbx2 - nbx1) * (nby2 - nby1)
    ox1_ref[:] = x1
    oy1_ref[:] = y1
    ox2_ref[:] = x2
    oy2_ref[:] = y2
    off_ref[:] = off

    flat_idx = (jax.lax.broadcasted_iota(jnp.int32, (_R, _C), 0) * _C
                + jax.lax.broadcasted_iota(jnp.int32, (_R, _C), 1))
    lane = jax.lax.broadcasted_iota(jnp.int32, (1, _C), 1)

    carry0 = _chunk_tournament((scores, flat_idx, x1, y1, x2, y2, off))

    def step(i, carry):
        best, _, ox1, oy1, ox2, oy2, goff = _final_tree(carry)
        kf = (best > 0.0).astype(jnp.float32)
        bx1 = ox1 + goff
        by1 = oy1 + goff
        bx2 = ox2 + goff
        by2 = oy2 + goff

        s = scores_ref[:]
        ix1 = jnp.maximum(bx1, nbx1_ref[:])
        iy1 = jnp.maximum(by1, nby1_ref[:])
        ix2 = jnp.minimum(bx2, nbx2_ref[:])
        iy2 = jnp.minimum(by2, nby2_ref[:])
        iw = jnp.maximum(ix2 - ix1, 0.0)
        ih = jnp.maximum(iy2 - iy1, 0.0)
        inter = iw * ih
        area_b = (bx2 - bx1) * (by2 - by1)
        iou = inter / (area_b + areas_ref[:] - inter + 1e-9)
        new_s = jnp.where(iou > _NMS_THRE, 0.0, s)
        scores_ref[:] = new_s

        row = (jnp.where(lane == 0, ox1 * kf, 0.0)
               + jnp.where(lane == 1, oy1 * kf, 0.0)
               + jnp.where(lane == 2, ox2 * kf, 0.0)
               + jnp.where(lane == 3, oy2 * kf, 0.0)
               + jnp.where(lane == 4, best * kf, 0.0))
        out_ref[pl.ds(i, 1), :] = row
        return _chunk_tournament((new_s, flat_idx, ox1_ref[:], oy1_ref[:],
                                  ox2_ref[:], oy2_ref[:], off_ref[:]))

    jax.lax.fori_loop(0, _MAX_DET, step, carry0)


def kernel(raw):
    rawp = jnp.pad(raw[0], ((0, _NPAD - _N_PRED), (0, 0)))
    rawt = rawp.T
    cx = rawt[0].reshape(_R, _C)
    cy = rawt[1].reshape(_R, _C)
    w = rawt[2].reshape(_R, _C)
    h = rawt[3].reshape(_R, _C)
    obj = rawt[4].reshape(_R, _C)
    cls = rawt[5:].reshape(_NUM_CLASSES, _R, _C)

    out = pl.pallas_call(
        _nms_body,
        out_shape=jax.ShapeDtypeStruct((_MAX_DET, _C), jnp.float32),
        scratch_shapes=[pltpu.VMEM((_R, _C), jnp.float32)] * 11,
    )(cx, cy, w, h, obj, cls)
    return out[:, :5]


# speedup vs baseline: 1.2398x; 1.2398x over previous
import jax
import jax.numpy as jnp
from jax.experimental import pallas as pl
from jax.experimental.pallas import tpu as pltpu

_NUM_CLASSES = 80
_TEST_CONF = 0.01
_NMS_THRE = 0.65
_MAX_DET = 200
_N_PRED = 20000
_R = 160
_C = 128
_NPAD = _R * _C
_CH = 8
_NCHUNK = _R // _CH
_BIG = 0x3FFFFFFF
_LOOP = 200


def _tournament(s, flat_idx):
    pairs = [(s[v * _CH:(v + 1) * _CH], flat_idx[v * _CH:(v + 1) * _CH])
             for v in range(_NCHUNK)]
    while len(pairs) > 1:
        nxt = []
        for j in range(0, len(pairs) - 1, 2):
            (ma, ia), (mb, ib) = pairs[j], pairs[j + 1]
            gt = mb > ma
            nxt.append((jnp.maximum(ma, mb), jnp.where(gt, ib, ia)))
        if len(pairs) % 2:
            nxt.append(pairs[-1])
        pairs = nxt
    return pairs[0]


def _nms_body(cx_ref, cy_ref, w_ref, h_ref, obj_ref, cls_ref, out_ref,
              scores_ref, nbx1_ref, nby1_ref, nbx2_ref, nby2_ref,
              areas_ref, off_ref):
    def cls_step(k, carry):
        m, am = carry
        v = cls_ref[k]
        gt = v > m
        am = jnp.where(gt, k, am)
        m = jnp.maximum(m, v)
        return m, am

    m0 = cls_ref[0]
    am0 = jnp.zeros((_R, _C), jnp.int32)
    cls_conf, cls_pred = jax.lax.fori_loop(1, _NUM_CLASSES, cls_step,
                                           (m0, am0), unroll=8)
    conf = obj_ref[:] * cls_conf
    scores = jnp.where(conf >= _TEST_CONF, conf, 0.0)
    off = cls_pred.astype(jnp.float32) * 4096.0

    cx = cx_ref[:]
    cy = cy_ref[:]
    w = w_ref[:]
    h = h_ref[:]
    x1 = cx - w / 2.0
    y1 = cy - h / 2.0
    x2 = cx + w / 2.0
    y2 = cy + h / 2.0
    nbx1 = x1 + off
    nby1 = y1 + off
    nbx2 = x2 + off
    nby2 = y2 + off

    scores_ref[:] = scores
    nbx1_ref[:] = nbx1
    nby1_ref[:] = nby1
    nbx2_ref[:] = nbx2
    nby2_ref[:] = nby2
    areas_ref[:] = (nbx2 - nbx1) * (nby2 - nby1)
    off_ref[:] = off

    flat_idx = (jax.lax.broadcasted_iota(jnp.int32, (_R, _C), 0) * _C
                + jax.lax.broadcasted_iota(jnp.int32, (_R, _C), 1))
    lane = jax.lax.broadcasted_iota(jnp.int32, (1, _C), 1)

    m8_0, i8_0 = _tournament(scores, flat_idx)

    def step(i, carry):
        m8, i8 = carry
        best = jnp.max(m8, axis=None, keepdims=True)
        idx = jnp.min(jnp.where(m8 == best, i8, _BIG))
        kf = (best > 0.0).astype(jnp.float32)
        r = idx // _C
        c = idx - r * _C

        rows = jnp.concatenate(
            [cx_ref[pl.ds(r, 1), :], cy_ref[pl.ds(r, 1), :],
             w_ref[pl.ds(r, 1), :], h_ref[pl.ds(r, 1), :],
             off_ref[pl.ds(r, 1), :]], axis=0)
        g = jnp.sum(jnp.where(lane == c, rows, 0.0), axis=1,
                    keepdims=True)
        gcx = g[0:1]
        gcy = g[1:2]
        gw = g[2:3]
        gh = g[3:4]
        goff = g[4:5]
        ox1 = gcx - gw / 2.0
        oy1 = gcy - gh / 2.0
        ox2 = gcx + gw / 2.0
        oy2 = gcy + gh / 2.0
        bx1 = ox1 + goff
        by1 = oy1 + goff
        bx2 = ox2 + goff
        by2 = oy2 + goff

        s = scores_ref[:]
        ix1 = jnp.maximum(bx1, nbx1_ref[:])
        iy1 = jnp.maximum(by1, nby1_ref[:])
        ix2 = jnp.minimum(bx2, nbx2_ref[:])
        iy2 = jnp.minimum(by2, nby2_ref[:])
        iw = jnp.maximum(ix2 - ix1, 0.0)
        ih = jnp.maximum(iy2 - iy1, 0.0)
        inter = iw * ih
        area_b = (bx2 - bx1) * (by2 - by1)
        iou = inter / (area_b + areas_ref[:] - inter + 1e-9)
        new_s = jnp.where(iou > _NMS_THRE, 0.0, s)
        scores_ref[:] = new_s

        row = (jnp.where(lane == 0, ox1 * kf, 0.0)
               + jnp.where(lane == 1, oy1 * kf, 0.0)
               + jnp.where(lane == 2, ox2 * kf, 0.0)
               + jnp.where(lane == 3, oy2 * kf, 0.0)
               + jnp.where(lane == 4, best * kf, 0.0))
        out_ref[pl.ds(i, 1), :] = row
        return _tournament(new_s, flat_idx)

    jax.lax.fori_loop(0, _LOOP, step, (m8_0, i8_0))


def kernel(raw):
    rawp = jnp.pad(raw[0], ((0, _NPAD - _N_PRED), (0, 0)))
    rawt = rawp.T
    cx = rawt[0].reshape(_R, _C)
    cy = rawt[1].reshape(_R, _C)
    w = rawt[2].reshape(_R, _C)
    h = rawt[3].reshape(_R, _C)
    obj = rawt[4].reshape(_R, _C)
    cls = rawt[5:].reshape(_NUM_CLASSES, _R, _C)

    out = pl.pallas_call(
        _nms_body,
        out_shape=jax.ShapeDtypeStruct((_MAX_DET, _C), jnp.float32),
        scratch_shapes=[pltpu.VMEM((_R, _C), jnp.float32)] * 7,
    )(cx, cy, w, h, obj, cls)
    return out[:, :5]
